# baseline (speedup 1.0000x reference)
import functools

import jax
import jax.numpy as jnp
from jax import lax
from jax.experimental import pallas as pl
from jax.experimental.pallas import tpu as pltpu
from jax.experimental.pallas import tpu_sc as plsc

_VOCAB = 1000000
_DIM = 64
_BATCH = 16384

_TAIL_START = 999936
_CHUNK_COLS = 256
_NCHUNK = _TAIL_START // _CHUNK_COLS
_NBATCH = 128
_BLIM = 16256
_SC_BOUNDARY = ((16 * _NCHUNK) // 32) * _CHUNK_COLS


def _make_gather(V, D, B):
  info = plsc.get_sparse_core_info()
  NC, NS = info.num_cores, info.num_subcores
  NW = NC * NS
  mesh = plsc.VectorSubcoreMesh(core_axis_name="c", subcore_axis_name="s")

  @functools.partial(
      pl.kernel,
      mesh=mesh,
      out_type=jax.ShapeDtypeStruct((NC * B, 2 * D), jnp.float32),
      scratch_types=[
          pltpu.VMEM((2048,), jnp.int32),
          pltpu.VMEM((B + 32,), jnp.int32),
          pltpu.VMEM((32,), jnp.int32),
          pltpu.VMEM((D, _CHUNK_COLS), jnp.float32),
          pltpu.VMEM((_NBATCH, 2 * D), jnp.float32),
          pltpu.VMEM((_NBATCH + 16,), jnp.int32),
          pltpu.VMEM((V - _TAIL_START, D), jnp.float32),
      ],
      compiler_params=pltpu.CompilerParams(needs_layout_passes=False),
  )
  def ka(tab_t, tail, idx_hbm, inter_hbm, idx_v, hlist_v, sub_v,
         chunk_v, stage_v, blist_v, tail_v):
    cc = lax.axis_index("c")
    ss = lax.axis_index("s")
    w = cc * NS + ss
    c0 = (w * _NCHUNK) // NW
    c1 = ((w + 1) * _NCHUNK) // NW
    lo = c0 * _CHUNK_COLS
    hi = jnp.where(w == NW - 1, V, c1 * _CHUNK_COLS)

    pltpu.sync_copy(tail, tail_v)

    iota = lax.iota(jnp.int32, 16)
    neg1 = jnp.full((16,), -1, jnp.int32)

    for q in range(_NBATCH // 16):
      blist_v[pl.ds(q * 16, 16)] = neg1

    def piece_body(p, cnt):
      pltpu.sync_copy(idx_hbm.at[pl.ds(p * 2048, 2048)], idx_v)

      def scan_body(g, cnt):
        v = idx_v[pl.ds(g * 16, 16)]
        m = (v >= lo) & (v < hi)
        bvec = iota + p * 2048 + g * 16
        packed = jnp.bitwise_or(jnp.left_shift(v - lo, 14), bvec)
        plsc.store_compressed(hlist_v.at[pl.ds(cnt, 16)], packed, mask=m)
        return cnt + plsc.all_reduce_population_count(m)[0]

      return lax.fori_loop(0, 2048 // 16, scan_body, cnt)

    nh = lax.fori_loop(0, B // 2048, piece_body, jnp.int32(0))
    hlist_v[pl.ds(nh, 16)] = neg1
    ngrp = (nh + 15) // 16

    def flush():
      pltpu.sync_copy(
          stage_v,
          inter_hbm.at[plsc.Indices(blist_v.at[pl.ds(0, _NBATCH)],
                                    ignored_value=-1)],
      )
      for q in range(_NBATCH // 16):
        blist_v[pl.ds(q * 16, 16)] = neg1

    def emit(col_fetch, bscal, cnt_out):
      slot = jnp.bitwise_and(cnt_out, _NBATCH - 1)
      for g3 in range(D // 16):
        stage_v[slot, pl.ds(g3 * 16, 16)] = col_fetch(g3)
      blist_v[pl.ds(slot, 16)] = jnp.full((16,), cc * B + bscal, jnp.int32)

      @pl.when(slot == _NBATCH - 1)
      def _():
        flush()

      return cnt_out + 1

    def chunk_body(c, cnt_out):
      col0 = pl.multiple_of(c * _CHUNK_COLS, _CHUNK_COLS)
      pltpu.sync_copy(tab_t.at[:, pl.ds(col0, _CHUNK_COLS)], chunk_v)

      plo = jnp.left_shift(col0 - lo, 14)
      phi = plo + (_CHUNK_COLS << 14)

      def grp_body(g2, cnt_out):
        hp = hlist_v[pl.ds(g2 * 16, 16)]
        m2 = (hp >= plo) & (hp < phi)
        plsc.store_compressed(sub_v.at[pl.ds(0, 16)], hp, mask=m2)
        mcount = plsc.all_reduce_population_count(m2)[0]

        def match_body(e, cnt_out):
          p2 = sub_v[pl.ds(e, 16)][0]
          col = jnp.right_shift(p2, 14) - (col0 - lo)
          bscal = jnp.bitwise_and(p2, 16383)
          colv = jnp.full((16,), col, jnp.int32)
          return emit(
              lambda g3: plsc.load_gather(chunk_v, [iota + g3 * 16, colv]),
              bscal, cnt_out)

        return lax.fori_loop(0, mcount, match_body, cnt_out)

      return lax.fori_loop(0, ngrp, grp_body, cnt_out)

    cnt_out = lax.fori_loop(c0, c1, chunk_body, jnp.int32(0))

    ptail = jnp.left_shift(_TAIL_START - lo, 14)

    def tail_grp_body(g2, cnt_out):
      hp = hlist_v[pl.ds(g2 * 16, 16)]
      m3 = hp >= ptail
      plsc.store_compressed(sub_v.at[pl.ds(0, 16)], hp, mask=m3)
      mcount = plsc.all_reduce_population_count(m3)[0]

      def match_body(e, cnt_out):
        p2 = sub_v[pl.ds(e, 16)][0]
        r = jnp.right_shift(p2, 14) - (_TAIL_START - lo)
        bscal = jnp.bitwise_and(p2, 16383)
        rv = jnp.full((16,), r, jnp.int32)
        return emit(
            lambda g3: plsc.load_gather(tail_v, [rv, iota + g3 * 16]),
            bscal, cnt_out)

      return lax.fori_loop(0, mcount, match_body, cnt_out)

    ntail_grp = jnp.where(w == NW - 1, ngrp, 0)
    cnt_out = lax.fori_loop(0, ntail_grp, tail_grp_body, cnt_out)

    blist_v[pl.ds(jnp.bitwise_and(cnt_out, _NBATCH - 1), 16)] = neg1
    flush()


  @functools.partial(
      pl.kernel,
      mesh=mesh,
      out_type=jax.ShapeDtypeStruct((D, B), jnp.float32),
      scratch_types=[
          pltpu.VMEM((B // NW,), jnp.int32),
          pltpu.VMEM((B // NW, 2 * D), jnp.float32),
          pltpu.VMEM((D, B // NW), jnp.float32),
      ],
      compiler_params=pltpu.CompilerParams(needs_layout_passes=False),
  )
  def kb(inter_hbm, idx_hbm, out_t_hbm, idxb_v, rcat_v, outst_v):
    cc = lax.axis_index("c")
    ss = lax.axis_index("s")
    w = cc * NS + ss
    n = B // NW
    base = w * n
    pltpu.sync_copy(idx_hbm.at[pl.ds(base, n)], idxb_v)

    iota = lax.iota(jnp.int32, 16)
    half = n // 2

    for h in range(2):
      hb = base + h * half
      pltpu.sync_copy(inter_hbm.at[pl.ds(hb, half)], rcat_v.at[pl.ds(0, half)])
      pltpu.sync_copy(inter_hbm.at[pl.ds(B + hb, half)],
                      rcat_v.at[pl.ds(half, half)])

      def grp_body(g, carry):
        own = idxb_v[pl.ds(h * half + g * 16, 16)] >= _SC_BOUNDARY
        rowv = iota + g * 16 + jnp.where(own, half, 0)
        for col in range(D):
          colv = jnp.full((16,), col, jnp.int32)
          outst_v[col, pl.ds(h * half + g * 16, 16)] = plsc.load_gather(
              rcat_v, [rowv, colv])
        return carry

      lax.fori_loop(0, half // 16, grp_body, jnp.int32(0))

    pltpu.sync_copy(outst_v, out_t_hbm.at[:, pl.ds(base, n)])

  return ka, kb


_ka, _kb = _make_gather(_VOCAB, _DIM, _BATCH)


@jax.jit
def kernel(indices, table):
  table_t = table.T
  tail = table[_TAIL_START:]
  idx = indices.astype(jnp.int32)
  inter = _ka(table_t, tail, idx)
  out_t = _kb(inter, idx)
  return out_t.T


# speedup vs baseline: 1.3125x; 1.3125x over previous
import functools

import jax
import jax.numpy as jnp
from jax import lax
from jax.experimental import pallas as pl
from jax.experimental.pallas import tpu as pltpu
from jax.experimental.pallas import tpu_sc as plsc

_VOCAB = 1000000
_DIM = 64
_BATCH = 16384

_TAIL_START = 999936
_CHUNK_COLS = 512
_NCHUNK = _TAIL_START // _CHUNK_COLS
_NBATCH = 128
_BLIM = 16256
_SC_BOUNDARY = ((16 * _NCHUNK) // 32) * _CHUNK_COLS


def _make_gather(V, D, B):
  info = plsc.get_sparse_core_info()
  NC, NS = info.num_cores, info.num_subcores
  NW = NC * NS
  mesh = plsc.VectorSubcoreMesh(core_axis_name="c", subcore_axis_name="s")

  @functools.partial(
      pl.kernel,
      mesh=mesh,
      out_type=jax.ShapeDtypeStruct((NC * B, 2 * D), jnp.float32),
      scratch_types=[
          pltpu.VMEM((2048,), jnp.int32),
          pltpu.VMEM((B + 32,), jnp.int32),
          pltpu.VMEM((32,), jnp.int32),
          pltpu.VMEM((D, _CHUNK_COLS), jnp.float32),
          pltpu.VMEM((_NBATCH, 2 * D), jnp.float32),
          pltpu.VMEM((_NBATCH + 16,), jnp.int32),
          pltpu.VMEM((V - _TAIL_START, D), jnp.float32),
      ],
      compiler_params=pltpu.CompilerParams(needs_layout_passes=False),
  )
  def ka(tab_t, tail, idx_hbm, inter_hbm, idx_v, hlist_v, sub_v,
         chunk_v, stage_v, blist_v, tail_v):
    cc = lax.axis_index("c")
    ss = lax.axis_index("s")
    w = cc * NS + ss
    c0 = (w * _NCHUNK) // NW
    c1 = ((w + 1) * _NCHUNK) // NW
    lo = c0 * _CHUNK_COLS
    hi = jnp.where(w == NW - 1, V, c1 * _CHUNK_COLS)

    pltpu.sync_copy(tail, tail_v)

    iota = lax.iota(jnp.int32, 16)
    neg1 = jnp.full((16,), -1, jnp.int32)

    for q in range(_NBATCH // 16):
      blist_v[pl.ds(q * 16, 16)] = neg1

    def piece_body(p, cnt):
      pltpu.sync_copy(idx_hbm.at[pl.ds(p * 2048, 2048)], idx_v)

      def scan_body(g, cnt):
        v = idx_v[pl.ds(g * 16, 16)]
        m = (v >= lo) & (v < hi)
        bvec = iota + p * 2048 + g * 16
        packed = jnp.bitwise_or(jnp.left_shift(v - lo, 14), bvec)
        plsc.store_compressed(hlist_v.at[pl.ds(cnt, 16)], packed, mask=m)
        return cnt + plsc.all_reduce_population_count(m)[0]

      return lax.fori_loop(0, 2048 // 16, scan_body, cnt)

    nh = lax.fori_loop(0, B // 2048, piece_body, jnp.int32(0))
    hlist_v[pl.ds(nh, 16)] = neg1
    ngrp = (nh + 15) // 16

    def flush():
      pltpu.sync_copy(
          stage_v,
          inter_hbm.at[plsc.Indices(blist_v.at[pl.ds(0, _NBATCH)],
                                    ignored_value=-1)],
      )
      for q in range(_NBATCH // 16):
        blist_v[pl.ds(q * 16, 16)] = neg1

    def emit(col_fetch, bscal, cnt_out):
      slot = jnp.bitwise_and(cnt_out, _NBATCH - 1)
      for g3 in range(D // 16):
        stage_v[slot, pl.ds(g3 * 16, 16)] = col_fetch(g3)
      blist_v[pl.ds(slot, 16)] = jnp.full((16,), cc * B + bscal, jnp.int32)

      @pl.when(slot == _NBATCH - 1)
      def _():
        flush()

      return cnt_out + 1

    def chunk_body(c, cnt_out):
      col0 = pl.multiple_of(c * _CHUNK_COLS, _CHUNK_COLS)
      pltpu.sync_copy(tab_t.at[:, pl.ds(col0, _CHUNK_COLS)], chunk_v)

      plo = jnp.left_shift(col0 - lo, 14)
      phi = plo + (_CHUNK_COLS << 14)

      def grp_body(g2, cnt_out):
        hp = hlist_v[pl.ds(g2 * 16, 16)]
        m2 = (hp >= plo) & (hp < phi)
        plsc.store_compressed(sub_v.at[pl.ds(0, 16)], hp, mask=m2)
        mcount = plsc.all_reduce_population_count(m2)[0]

        def match_body(e, cnt_out):
          p2 = sub_v[pl.ds(e, 16)][0]
          col = jnp.right_shift(p2, 14) - (col0 - lo)
          bscal = jnp.bitwise_and(p2, 16383)
          colv = jnp.full((16,), col, jnp.int32)
          return emit(
              lambda g3: plsc.load_gather(chunk_v, [iota + g3 * 16, colv]),
              bscal, cnt_out)

        return lax.fori_loop(0, mcount, match_body, cnt_out)

      return lax.fori_loop(0, ngrp, grp_body, cnt_out)

    cnt_out = lax.fori_loop(c0, c1, chunk_body, jnp.int32(0))

    ptail = jnp.left_shift(_TAIL_START - lo, 14)

    def tail_grp_body(g2, cnt_out):
      hp = hlist_v[pl.ds(g2 * 16, 16)]
      m3 = hp >= ptail
      plsc.store_compressed(sub_v.at[pl.ds(0, 16)], hp, mask=m3)
      mcount = plsc.all_reduce_population_count(m3)[0]

      def match_body(e, cnt_out):
        p2 = sub_v[pl.ds(e, 16)][0]
        r = jnp.right_shift(p2, 14) - (_TAIL_START - lo)
        bscal = jnp.bitwise_and(p2, 16383)
        rv = jnp.full((16,), r, jnp.int32)
        return emit(
            lambda g3: plsc.load_gather(tail_v, [rv, iota + g3 * 16]),
            bscal, cnt_out)

      return lax.fori_loop(0, mcount, match_body, cnt_out)

    ntail_grp = jnp.where(w == NW - 1, ngrp, 0)
    cnt_out = lax.fori_loop(0, ntail_grp, tail_grp_body, cnt_out)

    blist_v[pl.ds(jnp.bitwise_and(cnt_out, _NBATCH - 1), 16)] = neg1
    flush()


  @functools.partial(
      pl.kernel,
      mesh=mesh,
      out_type=jax.ShapeDtypeStruct((D, B), jnp.float32),
      scratch_types=[
          pltpu.VMEM((B // NW,), jnp.int32),
          pltpu.VMEM((B // NW, 2 * D), jnp.float32),
          pltpu.VMEM((D, B // NW), jnp.float32),
      ],
      compiler_params=pltpu.CompilerParams(needs_layout_passes=False),
  )
  def kb(inter_hbm, idx_hbm, out_t_hbm, idxb_v, rcat_v, outst_v):
    cc = lax.axis_index("c")
    ss = lax.axis_index("s")
    w = cc * NS + ss
    n = B // NW
    base = w * n
    pltpu.sync_copy(idx_hbm.at[pl.ds(base, n)], idxb_v)

    iota = lax.iota(jnp.int32, 16)
    half = n // 2

    for h in range(2):
      hb = base + h * half
      pltpu.sync_copy(inter_hbm.at[pl.ds(hb, half)], rcat_v.at[pl.ds(0, half)])
      pltpu.sync_copy(inter_hbm.at[pl.ds(B + hb, half)],
                      rcat_v.at[pl.ds(half, half)])

      def grp_body(g, carry):
        own = idxb_v[pl.ds(h * half + g * 16, 16)] >= _SC_BOUNDARY
        rowv = iota + g * 16 + jnp.where(own, half, 0)
        for col in range(D):
          colv = jnp.full((16,), col, jnp.int32)
          outst_v[col, pl.ds(h * half + g * 16, 16)] = plsc.load_gather(
              rcat_v, [rowv, colv])
        return carry

      lax.fori_loop(0, half // 16, grp_body, jnp.int32(0))

    pltpu.sync_copy(outst_v, out_t_hbm.at[:, pl.ds(base, n)])

  return ka, kb


_ka, _kb = _make_gather(_VOCAB, _DIM, _BATCH)


@jax.jit
def kernel(indices, table):
  table_t = table.T
  tail = table[_TAIL_START:]
  idx = indices.astype(jnp.int32)
  inter = _ka(table_t, tail, idx)
  out_t = _kb(inter, idx)
  return out_t.T


# speedup vs baseline: 1.5099x; 1.1504x over previous
import functools

import jax
import jax.numpy as jnp
from jax import lax
from jax.experimental import pallas as pl
from jax.experimental.pallas import tpu as pltpu
from jax.experimental.pallas import tpu_sc as plsc

_VOCAB = 1000000
_DIM = 64
_BATCH = 16384

_TAIL_START = 999936
_CHUNK_COLS = 512
_NCHUNK = _TAIL_START // _CHUNK_COLS
_NSLOT = 64


def _make_gather(V, D, B):
  info = plsc.get_sparse_core_info()
  NC, NS = info.num_cores, info.num_subcores
  NW = NC * NS
  mesh = plsc.VectorSubcoreMesh(core_axis_name="c", subcore_axis_name="s")

  @functools.partial(
      pl.kernel,
      mesh=mesh,
      out_type=jax.ShapeDtypeStruct((B, D), jnp.float32),
      scratch_types=[
          pltpu.VMEM((B,), jnp.int32),
          pltpu.VMEM((B + 32,), jnp.int32),
          pltpu.VMEM((B + 32,), jnp.int32),
          pltpu.VMEM((32,), jnp.int32),
          pltpu.VMEM((32,), jnp.int32),
          pltpu.VMEM((D, _CHUNK_COLS), jnp.float32),
          pltpu.VMEM((_NSLOT, D), jnp.float32),
          pltpu.VMEM((V - _TAIL_START, D), jnp.float32),
          pltpu.SemaphoreType.DMA,
      ],
      compiler_params=pltpu.CompilerParams(needs_layout_passes=False),
  )
  def k(tab_t, tail, idx_hbm, out_hbm, idx_v, hidx_v, hb_v, sub_v, subb_v,
        chunk_v, stage_v, tail_v, osem):
    w = lax.axis_index("s") * NC + lax.axis_index("c")
    c0 = (w * _NCHUNK) // NW
    c1 = ((w + 1) * _NCHUNK) // NW
    lo = c0 * _CHUNK_COLS
    hi = jnp.where(w == NW - 1, V, c1 * _CHUNK_COLS)

    pltpu.sync_copy(idx_hbm, idx_v)
    pltpu.sync_copy(tail, tail_v)

    iota = lax.iota(jnp.int32, 16)

    def scan_body(g, cnt):
      v = idx_v[pl.ds(g * 16, 16)]
      m = (v >= lo) & (v < hi)
      bvec = iota + g * 16
      plsc.store_compressed(hidx_v.at[pl.ds(cnt, 16)], v, mask=m)
      plsc.store_compressed(hb_v.at[pl.ds(cnt, 16)], bvec, mask=m)
      return cnt + plsc.all_reduce_population_count(m)[0]

    nh = lax.fori_loop(0, B // 16, scan_body, jnp.int32(0))
    hidx_v[pl.ds(nh, 16)] = jnp.full((16,), -1, jnp.int32)
    ngrp = (nh + 15) // 16

    def emit(col_fetch, bscal, cnt_out):
      slot = jnp.bitwise_and(cnt_out, _NSLOT - 1)

      @pl.when(cnt_out >= _NSLOT)
      def _():
        pltpu.make_async_copy(stage_v.at[0], out_hbm.at[0], osem).wait()

      for g3 in range(D // 16):
        stage_v[slot, pl.ds(g3 * 16, 16)] = col_fetch(g3)
      pltpu.async_copy(stage_v.at[slot], out_hbm.at[bscal], osem)
      return cnt_out + 1

    def chunk_body(c, cnt_out):
      col0 = pl.multiple_of(c * _CHUNK_COLS, _CHUNK_COLS)
      pltpu.sync_copy(tab_t.at[:, pl.ds(col0, _CHUNK_COLS)], chunk_v)

      def grp_body(g2, cnt_out):
        hv = hidx_v[pl.ds(g2 * 16, 16)]
        bv = hb_v[pl.ds(g2 * 16, 16)]
        m2 = (hv >= col0) & (hv < col0 + _CHUNK_COLS)
        plsc.store_compressed(sub_v.at[pl.ds(0, 16)], hv, mask=m2)
        plsc.store_compressed(subb_v.at[pl.ds(0, 16)], bv, mask=m2)
        mcount = plsc.all_reduce_population_count(m2)[0]

        def match_body(e, cnt_out):
          col = sub_v[pl.ds(e, 16)][0] - col0
          bscal = subb_v[pl.ds(e, 16)][0]
          colv = jnp.full((16,), col, jnp.int32)
          return emit(
              lambda g3: plsc.load_gather(chunk_v, [iota + g3 * 16, colv]),
              bscal, cnt_out)

        return lax.fori_loop(0, mcount, match_body, cnt_out)

      return lax.fori_loop(0, ngrp, grp_body, cnt_out)

    cnt_out = lax.fori_loop(c0, c1, chunk_body, jnp.int32(0))

    def tail_grp_body(g2, cnt_out):
      hv = hidx_v[pl.ds(g2 * 16, 16)]
      bv = hb_v[pl.ds(g2 * 16, 16)]
      m3 = hv >= _TAIL_START
      plsc.store_compressed(sub_v.at[pl.ds(0, 16)], hv, mask=m3)
      plsc.store_compressed(subb_v.at[pl.ds(0, 16)], bv, mask=m3)
      mcount = plsc.all_reduce_population_count(m3)[0]

      def match_body(e, cnt_out):
        r = sub_v[pl.ds(e, 16)][0] - _TAIL_START
        bscal = subb_v[pl.ds(e, 16)][0]
        rv = jnp.full((16,), r, jnp.int32)
        return emit(
            lambda g3: plsc.load_gather(tail_v, [rv, iota + g3 * 16]),
            bscal, cnt_out)

      return lax.fori_loop(0, mcount, match_body, cnt_out)

    ntail_grp = jnp.where(w == NW - 1, ngrp, 0)
    cnt_out = lax.fori_loop(0, ntail_grp, tail_grp_body, cnt_out)

    def drain_body(_, x):
      pltpu.make_async_copy(stage_v.at[0], out_hbm.at[0], osem).wait()
      return x

    lax.fori_loop(0, jnp.minimum(cnt_out, _NSLOT), drain_body, jnp.int32(0))

  return k


_scan_gather = _make_gather(_VOCAB, _DIM, _BATCH)


@jax.jit
def kernel(indices, table):
  table_t = table.T
  tail = table[_TAIL_START:]
  return _scan_gather(table_t, tail, indices.astype(jnp.int32))
